# baseline (speedup 1.0000x reference)
import functools

import jax
import jax.numpy as jnp
from jax import lax
from jax.experimental import pallas as pl
from jax.experimental.pallas import tpu as pltpu
from jax.experimental.pallas import tpu_sc as plsc

CONTEXT = 8192
DIM = 1024


def kernel(table):
    mesh = plsc.VectorSubcoreMesh(core_axis_name="c", subcore_axis_name="s")
    num_workers = mesh.num_cores * mesh.num_subcores
    rows_per_w = CONTEXT // num_workers

    @functools.partial(
        pl.kernel,
        out_type=jax.ShapeDtypeStruct((CONTEXT, DIM), jnp.float32),
        mesh=mesh,
        scratch_types=[pltpu.SemaphoreType.DMA],
    )
    def copy_rows(table_hbm, out_hbm, sem):
        wid = lax.axis_index("s") * mesh.num_cores + lax.axis_index("c")
        base = wid * rows_per_w
        pltpu.async_copy(
            table_hbm.at[pl.ds(base, rows_per_w)],
            out_hbm.at[pl.ds(base, rows_per_w)],
            sem,
        ).wait()

    return copy_rows(table)


# speedup vs baseline: 24.4481x; 24.4481x over previous
import functools

import jax
import jax.numpy as jnp
from jax import lax
from jax.experimental import pallas as pl
from jax.experimental.pallas import tpu as pltpu
from jax.experimental.pallas import tpu_sc as plsc

CONTEXT = 8192
DIM = 1024


NBUF = 2
CHUNK = 32


def kernel(table):
    mesh = plsc.VectorSubcoreMesh(core_axis_name="c", subcore_axis_name="s")
    num_workers = mesh.num_cores * mesh.num_subcores
    rows_per_w = CONTEXT // num_workers
    nchunks = rows_per_w // CHUNK

    @functools.partial(
        pl.kernel,
        out_type=jax.ShapeDtypeStruct((CONTEXT, DIM), jnp.float32),
        mesh=mesh,
        scratch_types=[
            pltpu.VMEM((NBUF, CHUNK, DIM), jnp.float32),
            pltpu.SemaphoreType.DMA,
            pltpu.SemaphoreType.DMA,
            pltpu.SemaphoreType.DMA,
            pltpu.SemaphoreType.DMA,
        ],
    )
    def copy_rows(table_hbm, out_hbm, bufs, in0, in1, out0, out1):
        wid = lax.axis_index("s") * mesh.num_cores + lax.axis_index("c")
        base = wid * rows_per_w
        in_sems = [in0, in1]
        out_sems = [out0, out1]

        def cp_in(g, b):
            return pltpu.make_async_copy(
                table_hbm.at[pl.ds(base + g * CHUNK, CHUNK)],
                bufs.at[b],
                in_sems[b],
            )

        def cp_out(g, b):
            return pltpu.make_async_copy(
                bufs.at[b],
                out_hbm.at[pl.ds(base + g * CHUNK, CHUNK)],
                out_sems[b],
            )

        for b in range(min(NBUF, nchunks)):
            cp_in(b, b).start()
        for g in range(nchunks):
            b = g % NBUF
            cp_in(g, b).wait()
            cp_out(g, b).start()
            nxt = g + NBUF
            if nxt < nchunks:
                cp_out(g, b).wait()
                cp_in(nxt, b).start()
        for g in range(max(0, nchunks - NBUF), nchunks):
            cp_out(g, g % NBUF).wait()

    return copy_rows(table)
